# Initial kernel scaffold; baseline (speedup 1.0000x reference)
import jax
import jax.numpy as jnp
from jax.experimental import pallas as pl


def kernel(x, k):
    raise NotImplementedError("write your pallas kernel here")


# speedup vs baseline: 1.5363x; 1.5363x over previous
import jax
import jax.numpy as jnp
from jax.experimental import pallas as pl

_NUM_SAMPLES = 512
_N_BISECT = 48


def _stats_kernel(x_ref, y_ref):
    x = x_ref[...]
    b, d = x.shape
    target = jnp.float32(d - d * 3 // 4)

    lo = jnp.min(x, axis=1, keepdims=True)
    hi = jnp.max(x, axis=1, keepdims=True)

    def body(_, carry):
        lo, hi = carry
        mid = lo * 0.5 + hi * 0.5
        c = jnp.sum((x <= mid).astype(jnp.float32), axis=1, keepdims=True)
        pred = c >= target
        return jnp.where(pred, lo, mid), jnp.where(pred, mid, hi)

    lo, hi = jax.lax.fori_loop(0, _N_BISECT, body, (lo, hi))
    mean = hi

    mu = jnp.sum(x, axis=1, keepdims=True) / d
    var = jnp.sum((x - mu) ** 2, axis=1, keepdims=True) / (d - 1)
    std = jnp.sqrt(var)

    x_norm = (x - mean) / std ** 0.3
    expo = jnp.clip(-x_norm / 0.001, -50.0, 50.0)
    y_ref[...] = 1.0 / (1.0 + jnp.exp(expo))


def _bcast_kernel(y_ref, o_ref):
    o_ref[...] = jnp.broadcast_to(y_ref[...][..., None], o_ref.shape)


def kernel(x, k):
    del k
    b, d = x.shape

    y = pl.pallas_call(
        _stats_kernel,
        out_shape=jax.ShapeDtypeStruct((b, d), x.dtype),
    )(x)

    bb, bd = 8, 256
    out = pl.pallas_call(
        _bcast_kernel,
        grid=(b // bb, d // bd),
        in_specs=[pl.BlockSpec((bb, bd), lambda i, j: (i, j))],
        out_specs=pl.BlockSpec((bb, bd, _NUM_SAMPLES), lambda i, j: (i, j, 0)),
        out_shape=jax.ShapeDtypeStruct((b, d, _NUM_SAMPLES), x.dtype),
    )(y)
    return out
